# Initial kernel scaffold; baseline (speedup 1.0000x reference)
import jax
import jax.numpy as jnp
from jax.experimental import pallas as pl


def kernel(inputs, weight):
    raise NotImplementedError("write your pallas kernel here")


# speedup vs baseline: 1.8448x; 1.8448x over previous
import functools

import jax
import jax.numpy as jnp
from jax import lax
from jax.experimental import pallas as pl
from jax.experimental.pallas import tpu as pltpu
from jax.experimental.pallas import tpu_sc as plsc

K = 8192
D = 32
N = 16384
COMMIT = 0.25

ROW_BLK = 512
COL_BLK = 2048


def _argmin_body(x_ref, w_ref, idx_ref, w2_ref):
    @pl.when(pl.program_id(0) == 0)
    def _():
        w = w_ref[...]
        w2_ref[...] = jnp.sum(w * w, axis=1)

    x = x_ref[...]
    x2 = jnp.sum(x * x, axis=1)
    colf = lax.broadcasted_iota(jnp.int32, (ROW_BLK, COL_BLK), 1
                                ).astype(jnp.float32)
    run_min = None
    run_idx = None
    for c in range(K // COL_BLK):
        w = w_ref[c * COL_BLK:(c + 1) * COL_BLK, :]
        w2 = w2_ref[c * COL_BLK:(c + 1) * COL_BLK]
        m = lax.dot_general(x, w, (((1,), (1,)), ((), ())),
                            preferred_element_type=jnp.float32)
        d = (x2[:, None] + w2[None, :]) - 2.0 * m
        cmin = jnp.min(d, axis=1)
        cidx = jnp.min(jnp.where(d == cmin[:, None], colf,
                                 jnp.float32(COL_BLK)),
                       axis=1) + jnp.float32(c * COL_BLK)
        if run_min is None:
            run_min, run_idx = cmin, cidx
        else:
            take = cmin < run_min
            run_idx = jnp.where(take, cidx, run_idx)
            run_min = jnp.minimum(cmin, run_min)
    idx_ref[...] = run_idx.astype(jnp.int32)


def _argmin_indices(xf, weight):
    return pl.pallas_call(
        _argmin_body,
        grid=(N // ROW_BLK,),
        in_specs=[
            pl.BlockSpec((ROW_BLK, D), lambda i: (i, 0)),
            pl.BlockSpec((K, D), lambda i: (0, 0)),
        ],
        out_specs=pl.BlockSpec((ROW_BLK,), lambda i: (i,)),
        out_shape=jax.ShapeDtypeStruct((N,), jnp.int32),
        scratch_shapes=[pltpu.VMEM((K,), jnp.float32)],
    )(xf, weight)


_NC = 2
_NS = 16
_NW = _NC * _NS
_BPW = N // _NW
_ICH = 128
_NCH = _BPW // _ICH


def _gather_hist_body(w_hbm, idx_hbm, q_hbm, cnt_hbm,
                      idx_v, rows_v, ones_v, zcnt_v, shared_cnt, sem):
    c = lax.axis_index("c")
    s = lax.axis_index("s")
    wid = c * _NS + s
    base = wid * _BPW

    pltpu.sync_copy(idx_hbm.at[pl.ds(wid * _NCH, _NCH), :], idx_v)

    copies = []
    for j in range(_NCH):
        copies.append(pltpu.async_copy(
            w_hbm.at[idx_v.at[j]],
            rows_v.at[pl.ds(j * _ICH, _ICH), :], sem))
    for cp in copies:
        cp.wait()
    pltpu.sync_copy(rows_v, q_hbm.at[pl.ds(base, _BPW)])

    zslice = K // _NS
    def _zero(i, _):
        zcnt_v[pl.ds(i * 16, 16)] = jnp.zeros((16,), jnp.float32)
        return 0
    lax.fori_loop(0, zslice // 16, _zero, 0)
    def _one(i, _):
        ones_v[pl.ds(i * 16, 16)] = jnp.ones((16,), jnp.float32)
        return 0
    lax.fori_loop(0, _ICH // 16, _one, 0)
    pltpu.sync_copy(zcnt_v, shared_cnt.at[pl.ds(s * zslice, zslice)])
    plsc.subcore_barrier()

    for j in range(_NCH):
        pltpu.sync_copy(ones_v, shared_cnt.at[idx_v.at[j]], add=True)
    plsc.subcore_barrier()

    pltpu.sync_copy(shared_cnt.at[pl.ds(s * zslice, zslice)], zcnt_v)
    pltpu.sync_copy(zcnt_v, cnt_hbm.at[wid])


@functools.cache
def _gather_hist_kernel():
    return pl.kernel(
        _gather_hist_body,
        out_type=(jax.ShapeDtypeStruct((N, D), jnp.float32),
                  jax.ShapeDtypeStruct((_NW, K // _NS), jnp.float32)),
        mesh=plsc.VectorSubcoreMesh(core_axis_name="c", subcore_axis_name="s"),
        scratch_types=[
            pltpu.VMEM((_NCH, _ICH), jnp.int32),
            pltpu.VMEM((_BPW, D), jnp.float32),
            pltpu.VMEM((_ICH,), jnp.float32),
            pltpu.VMEM((K // _NS,), jnp.float32),
            pltpu.VMEM_SHARED((K,), jnp.float32),
            pltpu.SemaphoreType.DMA,
        ],
        compiler_params=pltpu.CompilerParams(use_tc_tiling_on_sc=False),
    )


_FR, _FC = (N * D) // 128, 128


def _final_body(x_ref, q_ref, cnt_ref, qst_ref, loss_ref, perp_ref):
    x = x_ref[...]
    q = q_ref[...].astype(jnp.bfloat16).astype(jnp.float32)
    diff = q - x
    qst_ref[...] = x + diff
    m = jnp.sum(diff * diff) / jnp.float32(N * D)
    loss_ref[...] = jnp.broadcast_to(m + COMMIT * m, (1, 1))
    counts = cnt_ref[:_NS, :] + cnt_ref[_NS:, :]
    p = counts / jnp.float32(N)
    ent = jnp.sum(p * jnp.log(p + 1e-10))
    perp_ref[...] = jnp.broadcast_to(jnp.exp(-ent), (1, 1))


def _finalize(xf, q, counts):
    return pl.pallas_call(
        _final_body,
        in_specs=[
            pl.BlockSpec((_FR, _FC), lambda: (0, 0)),
            pl.BlockSpec((_FR, _FC), lambda: (0, 0)),
            pl.BlockSpec((_NW, K // _NS), lambda: (0, 0)),
        ],
        out_specs=[
            pl.BlockSpec((_FR, _FC), lambda: (0, 0)),
            pl.BlockSpec((1, 1), lambda: (0, 0)),
            pl.BlockSpec((1, 1), lambda: (0, 0)),
        ],
        out_shape=[
            jax.ShapeDtypeStruct((_FR, _FC), jnp.float32),
            jax.ShapeDtypeStruct((1, 1), jnp.float32),
            jax.ShapeDtypeStruct((1, 1), jnp.float32),
        ],
    )(xf, q, counts)


def kernel(inputs, weight):
    xf = inputs.reshape(N, D)
    idx = _argmin_indices(xf, weight)
    q, counts = _gather_hist_kernel()(weight, idx.reshape(N // _ICH, _ICH))
    qst, loss, perp = _finalize(xf.reshape(_FR, _FC), q.reshape(_FR, _FC),
                                counts)
    return (qst.reshape(inputs.shape), loss[0, 0], perp[0, 0])
